# Initial kernel scaffold; baseline (speedup 1.0000x reference)
import jax
import jax.numpy as jnp
from jax.experimental import pallas as pl


def kernel(pillar_features, voxel_coords, batch_size):
    raise NotImplementedError("write your pallas kernel here")


# speedup vs baseline: 4.0708x; 4.0708x over previous
import functools

import jax
import jax.numpy as jnp
from jax import lax
from jax.experimental import pallas as pl
from jax.experimental.pallas import tpu as pltpu
from jax.experimental.pallas import tpu_sc as plsc

_NX = 512
_NY = 512
_NZ = 1
_C = 64
_B = 4
_P = 80000
_CELLS = _B * _NZ * _NY * _NX
_NCORES = 2
_NSUB = 16
_NW = _NCORES * _NSUB
_CPW = _CELLS // _NW
_PAD_ROWS = 64
_CAN_ROWS = _CELLS + _PAD_ROWS
_CHUNK = 2000
_NCHUNK = _P // _CHUNK
_INNER = _CHUNK // 16
_BLK = 128
_CAP = _CPW + _BLK
_ZCH = 2048
_NZD = _CPW // _ZCH


def _sc_scatter_body(pf_hbm, coords_hbm, canvas_hbm,
                     cbuf0, cbuf1, map_ref, rows, cellblk, pblk,
                     scr, zbuf, sem_zero, dsem, sem_g, sem_s):
    cid = lax.axis_index("c")
    sid = lax.axis_index("s")
    wid = sid * _NCORES + cid
    lo = wid * _CPW
    iota = lax.iota(jnp.int32, 16)
    zeros16 = jnp.zeros((16,), jnp.float32)
    neg16 = jnp.full((16,), -1, jnp.int32)

    def _zrow(k, _):
        r = k // 4
        g = lax.rem(k, 4) * 16
        rows[r, pl.ds(g, 16)] = zeros16
        return 0
    lax.fori_loop(0, _BLK * _C // 16, _zrow, 0)

    @pl.when(sid == 0)
    def _():
        for k in range(_ZCH // _BLK):
            pltpu.sync_copy(rows, zbuf.at[pl.ds(k * _BLK, _BLK)])
    plsc.subcore_barrier()

    def _zfire(k, _):
        pltpu.make_async_copy(
            zbuf, canvas_hbm.at[pl.ds(lo + k * _ZCH, _ZCH)], sem_zero
        ).start()
        return 0
    lax.fori_loop(0, _NZD, _zfire, 0)

    scr[pl.ds(16, 16)] = neg16

    def _minit(g, _):
        map_ref[pl.ds(g * 16, 16)] = neg16
        return 0
    lax.fori_loop(0, _CPW // 16, _minit, 0)

    iota4 = iota * 4

    def _cstart(c, buf, sem):
        pltpu.make_async_copy(
            coords_hbm.at[pl.ds(c * _CHUNK * 4, _CHUNK * 4)], buf, sem
        ).start()

    def _cwait(c, buf, sem):
        pltpu.make_async_copy(
            coords_hbm.at[pl.ds(c * _CHUNK * 4, _CHUNK * 4)], buf, sem
        ).wait()

    def _scan_chunk(c, buf):
        def _inner(i, _):
            idx0 = iota4 + i * 64
            bv = plsc.load_gather(buf, [idx0])
            zv = plsc.load_gather(buf, [idx0 + 1])
            yv = plsc.load_gather(buf, [idx0 + 2])
            xv = plsc.load_gather(buf, [idx0 + 3])
            f = ((bv * _NZ + zv) * _NY + yv) * _NX + xv
            key = f * 16 + iota
            pvec = c * _CHUNK + i * 16 + iota
            ks, ps = plsc.sort_key_val(key, pvec)
            cells = lax.shift_right_logical(ks, 4)
            scr[pl.ds(0, 16)] = cells
            nxt = plsc.load_gather(scr, [iota + 1])
            lastm = cells != nxt
            locr = cells - lo
            inr = (locr >= 0) & (locr < _CPW)
            m = lastm & inr
            loc_safe = jnp.where(inr, locr, 0)
            plsc.store_scatter(map_ref, [loc_safe], ps, mask=m)
            return 0
        lax.fori_loop(0, _INNER, _inner, 0)

    _cstart(0, cbuf0, dsem.at[0])

    def _pair(q, _):
        c0 = q * 2
        _cwait(c0, cbuf0, dsem.at[0])
        _cstart(c0 + 1, cbuf1, dsem.at[1])
        _scan_chunk(c0, cbuf0)
        _cwait(c0 + 1, cbuf1, dsem.at[1])

        @pl.when(c0 + 2 < _NCHUNK)
        def _():
            _cstart(c0 + 2, cbuf0, dsem.at[0])
        _scan_chunk(c0 + 1, cbuf1)
        return 0
    lax.fori_loop(0, _NCHUNK // 2, _pair, 0)

    def _comp(g, cnt):
        w16 = map_ref[pl.ds(g * 16, 16)]
        m = w16 >= 0
        mi = m.astype(jnp.int32)
        pos = cnt + plsc.cumsum(mi) - 1
        pos_safe = jnp.where(m, pos, 0)
        packed = (g * 16 + iota) * 131072 + w16
        plsc.store_scatter(map_ref, [pos_safe], packed, mask=m)
        return cnt + jnp.sum(mi)
    n = lax.fori_loop(0, _CPW // 16, _comp, 0)

    def _zdrain(k, _):
        pltpu.make_async_copy(
            zbuf, canvas_hbm.at[pl.ds(lo + k * _ZCH, _ZCH)], sem_zero
        ).wait()
        return 0
    lax.fori_loop(0, _NZD, _zdrain, 0)

    nblk = (n + _BLK - 1) // _BLK

    def _eblk(j, _):
        base = j * _BLK
        for t in range(_BLK // 16):
            v16 = map_ref[pl.ds(base + t * 16, 16)]
            idxv = base + t * 16 + iota
            valid = idxv < n
            lcell = jnp.bitwise_and(
                lax.shift_right_logical(v16, 17), 0x7FFF)
            pval = jnp.bitwise_and(v16, 0x1FFFF)
            dcell = _CELLS + lax.rem(wid * 2 + t + iota, _PAD_ROWS)
            dp = lax.rem(wid * 16 + iota, _P)
            cellblk[pl.ds(t * 16, 16)] = jnp.where(valid, lo + lcell, dcell)
            pblk[pl.ds(t * 16, 16)] = jnp.where(valid, pval, dp)
        hg = pltpu.make_async_copy(pf_hbm.at[pblk], rows, sem_g)
        hg.start()
        hg.wait()
        hs = pltpu.make_async_copy(rows, canvas_hbm.at[cellblk], sem_s)
        hs.start()
        hs.wait()
        return 0
    lax.fori_loop(0, nblk, _eblk, 0)


@jax.jit
def _sc_scatter(pf, coords_flat):
    mesh = plsc.VectorSubcoreMesh(
        core_axis_name="c", subcore_axis_name="s",
        num_cores=_NCORES, num_subcores=_NSUB,
    )
    return pl.kernel(
        _sc_scatter_body,
        out_type=jax.ShapeDtypeStruct((_CAN_ROWS, _C), jnp.float32),
        mesh=mesh,
        compiler_params=pltpu.CompilerParams(
            needs_layout_passes=False, use_tc_tiling_on_sc=False
        ),
        scratch_types=[
            pltpu.VMEM((_CHUNK * 4,), jnp.int32),
            pltpu.VMEM((_CHUNK * 4,), jnp.int32),
            pltpu.VMEM((_CPW,), jnp.int32),
            pltpu.VMEM((_BLK, _C), jnp.float32),
            pltpu.VMEM((_BLK,), jnp.int32),
            pltpu.VMEM((_BLK,), jnp.int32),
            pltpu.VMEM((32,), jnp.int32),
            pltpu.VMEM_SHARED((_ZCH, _C), jnp.float32),
            pltpu.SemaphoreType.DMA,
            pltpu.SemaphoreType.DMA((2,)),
            pltpu.SemaphoreType.DMA,
            pltpu.SemaphoreType.DMA,
        ],
    )(pf, coords_flat)


def _tp_body(in_ref, out_ref):
    out_ref[...] = jnp.swapaxes(in_ref[...], 0, 1)[None]


@jax.jit
def _tc_transpose(canvas):
    return pl.pallas_call(
        _tp_body,
        out_shape=jax.ShapeDtypeStruct((_B, _C, _NY * _NX), jnp.float32),
        grid=(_CELLS // 512,),
        in_specs=[pl.BlockSpec((512, _C), lambda j: (j, 0))],
        out_specs=pl.BlockSpec((1, _C, 512), lambda j: (j // 512, 0, j % 512)),
    )(canvas)


def kernel(pillar_features, voxel_coords, batch_size):
    del batch_size
    coords_flat = voxel_coords.astype(jnp.int32).reshape(-1)
    canvas = _sc_scatter(pillar_features, coords_flat)
    out = _tc_transpose(canvas)
    return out.reshape(_B, _C * _NZ, _NY, _NX)


# speedup vs baseline: 16.0263x; 3.9369x over previous
import functools

import jax
import jax.numpy as jnp
from jax import lax
from jax.experimental import pallas as pl
from jax.experimental.pallas import tpu as pltpu
from jax.experimental.pallas import tpu_sc as plsc

_NX = 512
_NY = 512
_NZ = 1
_C = 64
_B = 4
_P = 80000
_CELLS = _B * _NZ * _NY * _NX
_NCORES = 2
_NSUB = 16
_NW = _NCORES * _NSUB
_CPW = _CELLS // _NW
_CHUNK = 1600
_NCHUNK = _P // _CHUNK
_INNER = _CHUNK // 16
_BLK = 128
_TCAP = 1024 + _BLK
_NT = 32


def _sc_body(pf_hbm, coords_hbm, out_hbm,
             cbuf0, cbuf1, map_ref, tbuf, rows, posblk, pblk, strips, scr,
             dsem, sem_g, sem_s):
    cid = lax.axis_index("c")
    sid = lax.axis_index("s")
    wid = sid * _NCORES + cid
    lo = wid * _CPW
    b = lax.shift_right_logical(wid, 3)
    ywbase = jnp.bitwise_and(wid, 7) * 64
    iota = lax.iota(jnp.int32, 16)
    zeros16 = jnp.zeros((16,), jnp.float32)
    neg16 = jnp.full((16,), -1, jnp.int32)

    def _sinit(k, _):
        c = k // 64
        r = lax.rem(k, 64) // 8
        g = lax.rem(k, 8) * 16
        strips[c, r, pl.ds(g, 16)] = zeros16
        return 0
    lax.fori_loop(0, _C * 8 * 128 // 16, _sinit, 0)

    scr[pl.ds(16, 16)] = neg16

    def _minit(g, _):
        map_ref[pl.ds(g * 16, 16)] = neg16
        return 0
    lax.fori_loop(0, _CPW // 16, _minit, 0)

    iota4 = iota * 4

    def _cstart(c, buf, sem):
        pltpu.make_async_copy(
            coords_hbm.at[pl.ds(c * _CHUNK * 4, _CHUNK * 4)], buf, sem
        ).start()

    def _cwait(c, buf, sem):
        pltpu.make_async_copy(
            coords_hbm.at[pl.ds(c * _CHUNK * 4, _CHUNK * 4)], buf, sem
        ).wait()

    def _scan_chunk(c, buf):
        def _inner(i, _):
            idx0 = iota4 + i * 64
            bv = plsc.load_gather(buf, [idx0])
            zv = plsc.load_gather(buf, [idx0 + 1])
            yv = plsc.load_gather(buf, [idx0 + 2])
            xv = plsc.load_gather(buf, [idx0 + 3])
            f = ((bv * _NZ + zv) * _NY + yv) * _NX + xv
            key = f * 16 + iota
            pvec = c * _CHUNK + i * 16 + iota
            ks, ps = plsc.sort_key_val(key, pvec)
            cells = lax.shift_right_logical(ks, 4)
            scr[pl.ds(0, 16)] = cells
            nxt = plsc.load_gather(scr, [iota + 1])
            lastm = cells != nxt
            locr = cells - lo
            inr = (locr >= 0) & (locr < _CPW)
            m = lastm & inr
            loc_safe = jnp.where(inr, locr, 0)
            plsc.store_scatter(map_ref, [loc_safe], ps, mask=m)
            return 0
        lax.fori_loop(0, _INNER, _inner, 0)

    _cstart(0, cbuf0, dsem.at[0])

    def _pair(q, _):
        c0 = q * 2
        _cwait(c0, cbuf0, dsem.at[0])
        _cstart(c0 + 1, cbuf1, dsem.at[1])
        _scan_chunk(c0, cbuf0)
        _cwait(c0 + 1, cbuf1, dsem.at[1])

        @pl.when(c0 + 2 < _NCHUNK)
        def _():
            _cstart(c0 + 2, cbuf0, dsem.at[0])
        _scan_chunk(c0 + 1, cbuf1)
        return 0
    lax.fori_loop(0, _NCHUNK // 2, _pair, 0)

    def _tile(t, _):
        ty = lax.shift_right_logical(t, 2)
        tx = jnp.bitwise_and(t, 3)
        l0t = ty * (8 * _NX) + tx * 128

        def _comp(rg, cnt):
            r = lax.shift_right_logical(rg, 3)
            g = jnp.bitwise_and(rg, 7)
            l16 = l0t + r * _NX + g * 16
            w16 = map_ref[pl.ds(l16, 16)]
            m = w16 >= 0
            mi = m.astype(jnp.int32)
            pos = cnt + plsc.cumsum(mi) - 1
            pos_safe = jnp.where(m, pos, 0)
            packed = (r * 128 + g * 16 + iota) * 131072 + w16
            plsc.store_scatter(tbuf, [pos_safe], packed, mask=m)
            return cnt + jnp.sum(mi)
        nt = lax.fori_loop(0, 64, _comp, 0)

        for j in range(_BLK // 16):
            idxp = nt + j * 16 + iota
            dp = lax.rem(wid * 16 + j * 16 + iota, _P)
            plsc.store_scatter(tbuf, [idxp], dp)

        nblk = (nt + _BLK - 1) // _BLK

        def _chunk(j, _):
            for tt in range(_BLK // 16):
                v16 = tbuf[pl.ds(j * _BLK + tt * 16, 16)]
                pblk[pl.ds(tt * 16, 16)] = jnp.bitwise_and(v16, 0x1FFFF)
                posblk[pl.ds(tt * 16, 16)] = lax.shift_right_logical(v16, 17)
            hg = pltpu.make_async_copy(pf_hbm.at[pblk], rows, sem_g)
            hg.start()
            hg.wait()
            ngrp = (jnp.minimum(nt - j * _BLK, _BLK) + 15) // 16

            def _grp(tt, _):
                widx = tt * 16 + iota
                pos16 = plsc.load_gather(posblk, [widx])
                yv = lax.shift_right_logical(pos16, 7)
                xv = jnp.bitwise_and(pos16, 127)
                valid = (j * _BLK + widx) < nt
                for c in range(_C):
                    vals = plsc.load_gather(rows, [widx, jnp.full((16,), c, jnp.int32)])
                    plsc.store_scatter(
                        strips, [jnp.full((16,), c, jnp.int32), yv, xv],
                        vals, mask=valid)
                return 0
            lax.fori_loop(0, ngrp, _grp, 0)
            return 0
        lax.fori_loop(0, nblk, _chunk, 0)

        dst = out_hbm.at[b, pl.ds(0, _C),
                         pl.ds(ywbase + ty * 8, 8), pl.ds(tx * 128, 128)]
        hs = pltpu.make_async_copy(strips, dst, sem_s)
        hs.start()
        hs.wait()

        ngz = (nt + 15) // 16

        def _unz(tt, _):
            widx = tt * 16 + iota
            v16 = plsc.load_gather(tbuf, [widx])
            pos16 = lax.shift_right_logical(v16, 17)
            yv = lax.shift_right_logical(pos16, 7)
            xv = jnp.bitwise_and(pos16, 127)
            valid = widx < nt
            for c in range(_C):
                plsc.store_scatter(
                    strips, [jnp.full((16,), c, jnp.int32), yv, xv],
                    zeros16, mask=valid)
            return 0
        lax.fori_loop(0, ngz, _unz, 0)
        return 0
    lax.fori_loop(0, _NT, _tile, 0)


@jax.jit
def _sc_scatter(pf128, coords_flat):
    mesh = plsc.VectorSubcoreMesh(
        core_axis_name="c", subcore_axis_name="s",
        num_cores=_NCORES, num_subcores=_NSUB,
    )
    return pl.kernel(
        _sc_body,
        out_type=jax.ShapeDtypeStruct((_B, _C, _NY, _NX), jnp.float32),
        mesh=mesh,
        compiler_params=pltpu.CompilerParams(needs_layout_passes=False),
        scratch_types=[
            pltpu.VMEM((_CHUNK * 4,), jnp.int32),
            pltpu.VMEM((_CHUNK * 4,), jnp.int32),
            pltpu.VMEM((_CPW,), jnp.int32),
            pltpu.VMEM((_TCAP,), jnp.int32),
            pltpu.VMEM((_BLK, 128), jnp.float32),
            pltpu.VMEM((_BLK,), jnp.int32),
            pltpu.VMEM((_BLK,), jnp.int32),
            pltpu.VMEM((_C, 8, 128), jnp.float32),
            pltpu.VMEM((32,), jnp.int32),
            pltpu.SemaphoreType.DMA((2,)),
            pltpu.SemaphoreType.DMA,
            pltpu.SemaphoreType.DMA,
        ],
    )(pf128, coords_flat)


def kernel(pillar_features, voxel_coords, batch_size):
    del batch_size
    coords_flat = voxel_coords.astype(jnp.int32).reshape(-1)
    pf128 = jnp.pad(pillar_features, ((0, 0), (0, 128 - _C)))
    return _sc_scatter(pf128, coords_flat)


# speedup vs baseline: 18.1508x; 1.1326x over previous
import functools

import jax
import jax.numpy as jnp
from jax import lax
from jax.experimental import pallas as pl
from jax.experimental.pallas import tpu as pltpu
from jax.experimental.pallas import tpu_sc as plsc

_NX = 512
_NY = 512
_NZ = 1
_C = 64
_B = 4
_P = 80000
_CELLS = _B * _NZ * _NY * _NX
_NCORES = 2
_NSUB = 16
_NW = _NCORES * _NSUB
_CPW = _CELLS // _NW
_CHUNK = 1600
_NCHUNK = _P // _CHUNK
_INNER = _CHUNK // 16
_BLK = 128
_TCAP = 1024 + _BLK
_NT = 32


def _sc_body(pf_hbm, bcol_hbm, ycol_hbm, xcol_hbm, out_hbm,
             bb0, yb0, xb0, bb1, yb1, xb1,
             map_ref, tbufa, tbufb, rows, posblk, pblk, strips,
             dsem, sem_g, sem_s):
    cid = lax.axis_index("c")
    sid = lax.axis_index("s")
    wid = sid * _NCORES + cid
    lo = wid * _CPW
    b = lax.shift_right_logical(wid, 3)
    ywbase = jnp.bitwise_and(wid, 7) * 64
    iota = lax.iota(jnp.int32, 16)
    zeros16 = jnp.zeros((16,), jnp.float32)
    neg16 = jnp.full((16,), -1, jnp.int32)

    def _sinit(k, _):
        c = k // 64
        r = lax.rem(k, 64) // 8
        g = lax.rem(k, 8) * 16
        strips[c, r, pl.ds(g, 16)] = zeros16
        return 0
    lax.fori_loop(0, _C * 8 * 128 // 16, _sinit, 0)

    def _minit(g, _):
        map_ref[pl.ds(g * 16, 16)] = neg16
        return 0
    lax.fori_loop(0, _CPW // 16, _minit, 0)

    def _cstart(c, bufs, sem):
        for col_hbm, buf in zip((bcol_hbm, ycol_hbm, xcol_hbm), bufs):
            pltpu.make_async_copy(
                col_hbm.at[pl.ds(c * _CHUNK, _CHUNK)], buf, sem
            ).start()

    def _cwait(c, bufs, sem):
        for col_hbm, buf in zip((bcol_hbm, ycol_hbm, xcol_hbm), bufs):
            pltpu.make_async_copy(
                col_hbm.at[pl.ds(c * _CHUNK, _CHUNK)], buf, sem
            ).wait()

    def _scan_chunk(c, bufs):
        bbuf, ybuf, xbuf = bufs

        def _inner(i, _):
            sl = pl.ds(i * 16, 16)
            bv = bbuf[sl]
            yv = ybuf[sl]
            xv = xbuf[sl]
            f = (bv * _NY + yv) * _NX + xv
            locr = f - lo
            inr = (locr >= 0) & (locr < _CPW)
            loc_safe = jnp.where(inr, locr, 0)
            pvec = c * _CHUNK + i * 16 + iota
            plsc.store_scatter(map_ref, [loc_safe], pvec, mask=inr)
            w = plsc.load_gather(map_ref, [loc_safe], mask=inr)
            fix = inr & (w < pvec)

            def _cond(fx):
                return jnp.sum(fx.astype(jnp.int32)) > 0

            def _body(fx):
                plsc.store_scatter(map_ref, [loc_safe], pvec, mask=fx)
                w2 = plsc.load_gather(map_ref, [loc_safe], mask=inr)
                return inr & (w2 < pvec)
            lax.while_loop(_cond, _body, fix)
            return 0
        lax.fori_loop(0, _INNER, _inner, 0)

    bufs0 = (bb0, yb0, xb0)
    bufs1 = (bb1, yb1, xb1)
    _cstart(0, bufs0, dsem.at[0])

    def _pair(q, _):
        c0 = q * 2
        _cwait(c0, bufs0, dsem.at[0])
        _cstart(c0 + 1, bufs1, dsem.at[1])
        _scan_chunk(c0, bufs0)
        _cwait(c0 + 1, bufs1, dsem.at[1])

        @pl.when(c0 + 2 < _NCHUNK)
        def _():
            _cstart(c0 + 2, bufs0, dsem.at[0])
        _scan_chunk(c0 + 1, bufs1)
        return 0
    lax.fori_loop(0, _NCHUNK // 2, _pair, 0)

    def _compact(t, tb):
        ty = lax.shift_right_logical(t, 2)
        tx = jnp.bitwise_and(t, 3)
        l0t = ty * (8 * _NX) + tx * 128

        def _comp(rg, cnt):
            r = lax.shift_right_logical(rg, 3)
            g = jnp.bitwise_and(rg, 7)
            l16 = l0t + r * _NX + g * 16
            w16 = map_ref[pl.ds(l16, 16)]
            m = w16 >= 0
            mi = m.astype(jnp.int32)
            pos = cnt + plsc.cumsum(mi) - 1
            pos_safe = jnp.where(m, pos, 0)
            packed = (r * 128 + g * 16 + iota) * 131072 + w16
            plsc.store_scatter(tb, [pos_safe], packed, mask=m)
            return cnt + jnp.sum(mi)
        nt = lax.fori_loop(0, 64, _comp, 0)

        for j in range(_BLK // 16):
            idxp = nt + j * 16 + iota
            dp = lax.rem(wid * 16 + j * 16 + iota, _P)
            plsc.store_scatter(tb, [idxp], dp)
        return nt

    def _unpack(j, tb):
        for tt in range(_BLK // 16):
            v16 = tb[pl.ds(j * _BLK + tt * 16, 16)]
            pblk[pl.ds(tt * 16, 16)] = jnp.bitwise_and(v16, 0x1FFFF)
            posblk[pl.ds(tt * 16, 16)] = lax.shift_right_logical(v16, 17)

    def _gather_start():
        pltpu.make_async_copy(pf_hbm.at[pblk], rows, sem_g).start()

    def _gather_wait():
        pltpu.make_async_copy(pf_hbm.at[pblk], rows, sem_g).wait()

    def _strip_dst(t):
        ty = lax.shift_right_logical(t, 2)
        tx = jnp.bitwise_and(t, 3)
        return out_hbm.at[b, pl.ds(0, _C),
                          pl.ds(ywbase + ty * 8, 8), pl.ds(tx * 128, 128)]

    def _scatter_chunk(j, nt):
        ngrp = (jnp.minimum(nt - j * _BLK, _BLK) + 15) // 16

        def _grp(tt, _):
            widx = tt * 16 + iota
            pos16 = plsc.load_gather(posblk, [widx])
            yv = lax.shift_right_logical(pos16, 7)
            xv = jnp.bitwise_and(pos16, 127)
            valid = (j * _BLK + widx) < nt
            for c in range(_C):
                vals = plsc.load_gather(
                    rows, [widx, jnp.full((16,), c, jnp.int32)])
                plsc.store_scatter(
                    strips, [jnp.full((16,), c, jnp.int32), yv, xv],
                    vals, mask=valid)
            return 0
        lax.fori_loop(0, ngrp, _grp, 0)

    def _unzero(tb, nt):
        ngz = (nt + 15) // 16

        def _unz(tt, _):
            widx = tt * 16 + iota
            v16 = plsc.load_gather(tb, [widx])
            pos16 = lax.shift_right_logical(v16, 17)
            yv = lax.shift_right_logical(pos16, 7)
            xv = jnp.bitwise_and(pos16, 127)
            valid = widx < nt
            for c in range(_C):
                plsc.store_scatter(
                    strips, [jnp.full((16,), c, jnp.int32), yv, xv],
                    zeros16, mask=valid)
            return 0
        lax.fori_loop(0, ngz, _unz, 0)

    def _chunks(t, tb, nt):
        nblk = jnp.maximum((nt + _BLK - 1) // _BLK, 1)

        def _chunk(j, _):
            @pl.when(j > 0)
            def _():
                _unpack(j, tb)
                _gather_start()
            _gather_wait()
            _scatter_chunk(j, nt)
            return 0
        lax.fori_loop(0, nblk, _chunk, 0)
        hs = pltpu.make_async_copy(strips, _strip_dst(t), sem_s)
        hs.start()

    def _pair_tile(q, ntb_prev):
        t0 = q * 2
        t1 = t0 + 1
        nta = _compact(t0, tbufa)
        _unpack(0, tbufa)
        _gather_start()

        @pl.when(q > 0)
        def _():
            pltpu.make_async_copy(strips, _strip_dst(t0), sem_s).wait()
            _unzero(tbufb, ntb_prev)
        _chunks(t0, tbufa, nta)
        ntb = _compact(t1, tbufb)
        _unpack(0, tbufb)
        _gather_start()
        pltpu.make_async_copy(strips, _strip_dst(t1), sem_s).wait()
        _unzero(tbufa, nta)
        _chunks(t1, tbufb, ntb)
        return ntb
    lax.fori_loop(0, _NT // 2, _pair_tile, 0)
    pltpu.make_async_copy(strips, _strip_dst(_NT - 1), sem_s).wait()


@jax.jit
def _sc_scatter(pf128, bcol, ycol, xcol):
    mesh = plsc.VectorSubcoreMesh(
        core_axis_name="c", subcore_axis_name="s",
        num_cores=_NCORES, num_subcores=_NSUB,
    )
    return pl.kernel(
        _sc_body,
        out_type=jax.ShapeDtypeStruct((_B, _C, _NY, _NX), jnp.float32),
        mesh=mesh,
        compiler_params=pltpu.CompilerParams(needs_layout_passes=False),
        scratch_types=[
            pltpu.VMEM((_CHUNK,), jnp.int32),
            pltpu.VMEM((_CHUNK,), jnp.int32),
            pltpu.VMEM((_CHUNK,), jnp.int32),
            pltpu.VMEM((_CHUNK,), jnp.int32),
            pltpu.VMEM((_CHUNK,), jnp.int32),
            pltpu.VMEM((_CHUNK,), jnp.int32),
            pltpu.VMEM((_CPW,), jnp.int32),
            pltpu.VMEM((_TCAP,), jnp.int32),
            pltpu.VMEM((_TCAP,), jnp.int32),
            pltpu.VMEM((_BLK, 128), jnp.float32),
            pltpu.VMEM((_BLK,), jnp.int32),
            pltpu.VMEM((_BLK,), jnp.int32),
            pltpu.VMEM((_C, 8, 128), jnp.float32),
            pltpu.SemaphoreType.DMA((2,)),
            pltpu.SemaphoreType.DMA,
            pltpu.SemaphoreType.DMA,
        ],
    )(pf128, bcol, ycol, xcol)


def kernel(pillar_features, voxel_coords, batch_size):
    del batch_size
    vc = voxel_coords.astype(jnp.int32)
    pf128 = jnp.pad(pillar_features, ((0, 0), (0, 128 - _C)))
    return _sc_scatter(pf128, vc[:, 0], vc[:, 2], vc[:, 3])


# speedup vs baseline: 24.3721x; 1.3428x over previous
import functools

import jax
import jax.numpy as jnp
from jax import lax
from jax.experimental import pallas as pl
from jax.experimental.pallas import tpu as pltpu
from jax.experimental.pallas import tpu_sc as plsc

_NX = 512
_NY = 512
_NZ = 1
_C = 64
_B = 4
_P = 80000
_CELLS = _B * _NZ * _NY * _NX
_NCORES = 2
_NSUB = 16
_NW = _NCORES * _NSUB
_CPW = _CELLS // _NW
_CHUNK = 1600
_NCHUNK = _P // _CHUNK
_INNER = _CHUNK // 16
_BLK = 128
_TCAP = 1024 + _BLK
_NT = 32


def _sc_body(pf_hbm, bcol_hbm, ycol_hbm, xcol_hbm, out_hbm,
             bb0, yb0, xb0, bb1, yb1, xb1,
             map_ref, tbufa, tbufb, rows, posblk, pblk, strips,
             dsem, sem_g, sem_s):
    cid = lax.axis_index("c")
    sid = lax.axis_index("s")
    wid = sid * _NCORES + cid
    lo = wid * _CPW
    b = lax.shift_right_logical(wid, 3)
    ywbase = jnp.bitwise_and(wid, 7) * 64
    iota = lax.iota(jnp.int32, 16)
    zeros16 = jnp.zeros((16,), jnp.float32)
    neg16 = jnp.full((16,), -1, jnp.int32)

    def _sinit(k, _):
        c = k // 64
        r = lax.rem(k, 64) // 8
        g = lax.rem(k, 8) * 16
        strips[c, r, pl.ds(g, 16)] = zeros16
        return 0
    lax.fori_loop(0, _C * 8 * 128 // 16, _sinit, 0)

    def _minit(g, _):
        map_ref[pl.ds(g * 16, 16)] = neg16
        return 0
    lax.fori_loop(0, _CPW // 16, _minit, 0)

    def _cstart(c, bufs, sem):
        for col_hbm, buf in zip((bcol_hbm, ycol_hbm, xcol_hbm), bufs):
            pltpu.make_async_copy(
                col_hbm.at[pl.ds(c * _CHUNK, _CHUNK)], buf, sem
            ).start()

    def _cwait(c, bufs, sem):
        for col_hbm, buf in zip((bcol_hbm, ycol_hbm, xcol_hbm), bufs):
            pltpu.make_async_copy(
                col_hbm.at[pl.ds(c * _CHUNK, _CHUNK)], buf, sem
            ).wait()

    def _scan_chunk(c, bufs):
        bbuf, ybuf, xbuf = bufs

        def _inner(i, _):
            sl = pl.ds(i * 16, 16)
            bv = bbuf[sl]
            yv = ybuf[sl]
            xv = xbuf[sl]
            f = (bv * _NY + yv) * _NX + xv
            locr = f - lo
            inr = (locr >= 0) & (locr < _CPW)
            loc_safe = jnp.where(inr, locr, 0)
            pvec = c * _CHUNK + i * 16 + iota
            plsc.store_scatter(map_ref, [loc_safe], pvec, mask=inr)
            return 0
        lax.fori_loop(0, _INNER, _inner, 0)

    bufs0 = (bb0, yb0, xb0)
    bufs1 = (bb1, yb1, xb1)
    _cstart(0, bufs0, dsem.at[0])

    def _pair(q, _):
        c0 = q * 2
        _cwait(c0, bufs0, dsem.at[0])
        _cstart(c0 + 1, bufs1, dsem.at[1])
        _scan_chunk(c0, bufs0)
        _cwait(c0 + 1, bufs1, dsem.at[1])

        @pl.when(c0 + 2 < _NCHUNK)
        def _():
            _cstart(c0 + 2, bufs0, dsem.at[0])
        _scan_chunk(c0 + 1, bufs1)
        return 0
    lax.fori_loop(0, _NCHUNK // 2, _pair, 0)

    def _compact(t, tb):
        ty = lax.shift_right_logical(t, 2)
        tx = jnp.bitwise_and(t, 3)
        l0t = ty * (8 * _NX) + tx * 128

        def _comp(rg, cnt):
            r = lax.shift_right_logical(rg, 3)
            g = jnp.bitwise_and(rg, 7)
            l16 = l0t + r * _NX + g * 16
            w16 = map_ref[pl.ds(l16, 16)]
            m = w16 >= 0
            mi = m.astype(jnp.int32)
            pos = cnt + plsc.cumsum(mi) - 1
            pos_safe = jnp.where(m, pos, 0)
            packed = (r * 128 + g * 16 + iota) * 131072 + w16
            plsc.store_scatter(tb, [pos_safe], packed, mask=m)
            return cnt + jnp.sum(mi)
        nt = lax.fori_loop(0, 64, _comp, 0)

        for j in range(_BLK // 16):
            idxp = nt + j * 16 + iota
            dp = lax.rem(wid * 16 + j * 16 + iota, _P)
            plsc.store_scatter(tb, [idxp], dp)
        return nt

    def _unpack(j, tb):
        for tt in range(_BLK // 16):
            v16 = tb[pl.ds(j * _BLK + tt * 16, 16)]
            pblk[pl.ds(tt * 16, 16)] = jnp.bitwise_and(v16, 0x1FFFF)
            posblk[pl.ds(tt * 16, 16)] = lax.shift_right_logical(v16, 17)

    def _gather_start():
        pltpu.make_async_copy(pf_hbm.at[pblk], rows, sem_g).start()

    def _gather_wait():
        pltpu.make_async_copy(pf_hbm.at[pblk], rows, sem_g).wait()

    def _strip_dst(t):
        ty = lax.shift_right_logical(t, 2)
        tx = jnp.bitwise_and(t, 3)
        return out_hbm.at[b, pl.ds(0, _C),
                          pl.ds(ywbase + ty * 8, 8), pl.ds(tx * 128, 128)]

    def _scatter_chunk(j, nt):
        ngrp = (jnp.minimum(nt - j * _BLK, _BLK) + 15) // 16

        def _grp(tt, _):
            widx = tt * 16 + iota
            pos16 = plsc.load_gather(posblk, [widx])
            yv = lax.shift_right_logical(pos16, 7)
            xv = jnp.bitwise_and(pos16, 127)
            valid = (j * _BLK + widx) < nt
            for c in range(_C):
                vals = plsc.load_gather(
                    rows, [widx, jnp.full((16,), c, jnp.int32)])
                plsc.store_scatter(
                    strips, [jnp.full((16,), c, jnp.int32), yv, xv],
                    vals, mask=valid)
            return 0
        lax.fori_loop(0, ngrp, _grp, 0)

    def _unzero(tb, nt):
        ngz = (nt + 15) // 16

        def _unz(tt, _):
            widx = tt * 16 + iota
            v16 = plsc.load_gather(tb, [widx])
            pos16 = lax.shift_right_logical(v16, 17)
            yv = lax.shift_right_logical(pos16, 7)
            xv = jnp.bitwise_and(pos16, 127)
            valid = widx < nt
            for c in range(_C):
                plsc.store_scatter(
                    strips, [jnp.full((16,), c, jnp.int32), yv, xv],
                    zeros16, mask=valid)
            return 0
        lax.fori_loop(0, ngz, _unz, 0)

    def _chunks(t, tb, nt):
        nblk = jnp.maximum((nt + _BLK - 1) // _BLK, 1)

        def _chunk(j, _):
            @pl.when(j > 0)
            def _():
                _unpack(j, tb)
                _gather_start()
            _gather_wait()
            _scatter_chunk(j, nt)
            return 0
        lax.fori_loop(0, nblk, _chunk, 0)
        hs = pltpu.make_async_copy(strips, _strip_dst(t), sem_s)
        hs.start()

    def _pair_tile(q, ntb_prev):
        t0 = q * 2
        t1 = t0 + 1
        nta = _compact(t0, tbufa)
        _unpack(0, tbufa)
        _gather_start()

        @pl.when(q > 0)
        def _():
            pltpu.make_async_copy(strips, _strip_dst(t0), sem_s).wait()
            _unzero(tbufb, ntb_prev)
        _chunks(t0, tbufa, nta)
        ntb = _compact(t1, tbufb)
        _unpack(0, tbufb)
        _gather_start()
        pltpu.make_async_copy(strips, _strip_dst(t1), sem_s).wait()
        _unzero(tbufa, nta)
        _chunks(t1, tbufb, ntb)
        return ntb
    lax.fori_loop(0, _NT // 2, _pair_tile, 0)
    pltpu.make_async_copy(strips, _strip_dst(_NT - 1), sem_s).wait()


@jax.jit
def _sc_scatter(pf128, bcol, ycol, xcol):
    mesh = plsc.VectorSubcoreMesh(
        core_axis_name="c", subcore_axis_name="s",
        num_cores=_NCORES, num_subcores=_NSUB,
    )
    return pl.kernel(
        _sc_body,
        out_type=jax.ShapeDtypeStruct((_B, _C, _NY, _NX), jnp.float32),
        mesh=mesh,
        compiler_params=pltpu.CompilerParams(needs_layout_passes=False),
        scratch_types=[
            pltpu.VMEM((_CHUNK,), jnp.int32),
            pltpu.VMEM((_CHUNK,), jnp.int32),
            pltpu.VMEM((_CHUNK,), jnp.int32),
            pltpu.VMEM((_CHUNK,), jnp.int32),
            pltpu.VMEM((_CHUNK,), jnp.int32),
            pltpu.VMEM((_CHUNK,), jnp.int32),
            pltpu.VMEM((_CPW,), jnp.int32),
            pltpu.VMEM((_TCAP,), jnp.int32),
            pltpu.VMEM((_TCAP,), jnp.int32),
            pltpu.VMEM((_BLK, 128), jnp.float32),
            pltpu.VMEM((_BLK,), jnp.int32),
            pltpu.VMEM((_BLK,), jnp.int32),
            pltpu.VMEM((_C, 8, 128), jnp.float32),
            pltpu.SemaphoreType.DMA((2,)),
            pltpu.SemaphoreType.DMA,
            pltpu.SemaphoreType.DMA,
        ],
    )(pf128, bcol, ycol, xcol)


def kernel(pillar_features, voxel_coords, batch_size):
    del batch_size
    vc = voxel_coords.astype(jnp.int32)
    pf128 = jnp.pad(pillar_features, ((0, 0), (0, 128 - _C)))
    return _sc_scatter(pf128, vc[:, 0], vc[:, 2], vc[:, 3])
